# baseline (speedup 1.0000x reference)
import functools

import jax
import jax.numpy as jnp
from jax.experimental import pallas as pl
from jax.experimental.pallas import tpu as pltpu

_BT = 1024
_E = 16
_NEG = -3.0e38


def _router_body(x_ref, w_ref, wout_ref, iout_ref):
    logits = jax.lax.dot_general(
        x_ref[...], w_ref[...],
        dimension_numbers=(((1,), (1,)), ((), ())),
        preferred_element_type=jnp.float32,
    )
    eidx = jax.lax.broadcasted_iota(jnp.int32, logits.shape, 1)
    m1 = jnp.max(logits, axis=1, keepdims=True)
    i1 = jnp.min(jnp.where(logits == m1, eidx, _E), axis=1, keepdims=True)
    masked = jnp.where(eidx == i1, _NEG, logits)
    m2 = jnp.max(masked, axis=1, keepdims=True)
    i2 = jnp.min(jnp.where(masked == m2, eidx, _E), axis=1, keepdims=True)
    z = jnp.exp(m2 - m1)
    w1 = 1.0 / (1.0 + z)
    w2 = z * w1
    wout_ref[...] = jnp.concatenate([w1, w2], axis=1)
    iout_ref[...] = jnp.concatenate([i1, i2], axis=1)


@jax.jit
def _route(x2d, W):
    nt = x2d.shape[0]
    grid = (nt // _BT,)
    return pl.pallas_call(
        _router_body,
        grid=grid,
        in_specs=[
            pl.BlockSpec((_BT, x2d.shape[1]), lambda i: (i, 0)),
            pl.BlockSpec((_E, x2d.shape[1]), lambda i: (0, 0)),
        ],
        out_specs=[
            pl.BlockSpec((_BT, 2), lambda i: (i, 0)),
            pl.BlockSpec((_BT, 2), lambda i: (i, 0)),
        ],
        out_shape=[
            jax.ShapeDtypeStruct((nt, 2), jnp.float32),
            jax.ShapeDtypeStruct((nt, 2), jnp.int32),
        ],
        compiler_params=pltpu.CompilerParams(
            dimension_semantics=("arbitrary",),
        ),
    )(x2d, W)


def kernel(x, W):
    B, T, D = x.shape
    w, i = _route(x.reshape(B * T, D), W)
    return w.reshape(B, T, 2), i.reshape(B, T, 2)


# speedup vs baseline: 1.0375x; 1.0375x over previous
import functools

import jax
import jax.numpy as jnp
from jax.experimental import pallas as pl
from jax.experimental.pallas import tpu as pltpu

_BT = 2048
_E = 16
_NEG = -3.0e38


def _router_body(x_ref, w_ref, wout_ref, iout_ref):
    logits = jax.lax.dot_general(
        x_ref[...], w_ref[...],
        dimension_numbers=(((1,), (1,)), ((), ())),
        preferred_element_type=jnp.float32,
    )
    eidx = jax.lax.broadcasted_iota(jnp.int32, logits.shape, 1)
    m1 = jnp.max(logits, axis=1, keepdims=True)
    i1 = jnp.min(jnp.where(logits == m1, eidx, _E), axis=1, keepdims=True)
    masked = jnp.where(eidx == i1, _NEG, logits)
    m2 = jnp.max(masked, axis=1, keepdims=True)
    i2 = jnp.min(jnp.where(masked == m2, eidx, _E), axis=1, keepdims=True)
    z = jnp.exp(m2 - m1)
    w1 = 1.0 / (1.0 + z)
    w2 = z * w1
    wout_ref[...] = jnp.concatenate([w1, w2], axis=1)
    iout_ref[...] = jnp.concatenate([i1, i2], axis=1)


@jax.jit
def _route(x2d, W):
    nt = x2d.shape[0]
    grid = (nt // _BT,)
    return pl.pallas_call(
        _router_body,
        grid=grid,
        in_specs=[
            pl.BlockSpec((_BT, x2d.shape[1]), lambda i: (i, 0)),
            pl.BlockSpec((_E, x2d.shape[1]), lambda i: (0, 0)),
        ],
        out_specs=[
            pl.BlockSpec((_BT, 2), lambda i: (i, 0)),
            pl.BlockSpec((_BT, 2), lambda i: (i, 0)),
        ],
        out_shape=[
            jax.ShapeDtypeStruct((nt, 2), jnp.float32),
            jax.ShapeDtypeStruct((nt, 2), jnp.int32),
        ],
        compiler_params=pltpu.CompilerParams(
            dimension_semantics=("arbitrary",),
        ),
    )(x2d, W)


def kernel(x, W):
    B, T, D = x.shape
    w, i = _route(x.reshape(B * T, D), W)
    return w.reshape(B, T, 2), i.reshape(B, T, 2)
